# baseline (speedup 1.0000x reference)
import functools

import jax
import jax.numpy as jnp
from jax import lax
from jax.experimental import pallas as pl
from jax.experimental.pallas import tpu as pltpu
from jax.experimental.pallas import tpu_sc as plsc

_B = 32768
_D = 1024
_NC = 2
_NS = 16
_NW = _NC * _NS
_BPW = _B // _NW
_CH = 16
_NCH = _BPW // _CH
_NB = 4
_NG = _NCH // _NB


@jax.jit
def _sc_gather(idx, table):
  mesh = plsc.VectorSubcoreMesh(core_axis_name="c", subcore_axis_name="s")

  @functools.partial(
      pl.kernel,
      out_type=jax.ShapeDtypeStruct((_B, _D), jnp.float32),
      mesh=mesh,
      scratch_types=[
          pltpu.VMEM((_NCH, _CH), jnp.int32),
      ]
      + [pltpu.VMEM((_CH, _D), jnp.float32)] * _NB
      + [pltpu.SemaphoreType.DMA] * (2 * _NB),
  )
  def k(idx_hbm, table_hbm, out_hbm, idx_v, *bufs):
    rows = bufs[:_NB]
    gsem = bufs[_NB:2 * _NB]
    ssem = bufs[2 * _NB:]
    wid = lax.axis_index("s") * _NC + lax.axis_index("c")
    base = wid * _BPW
    pltpu.sync_copy(idx_hbm.at[wid], idx_v)

    for b in range(_NB):
      pltpu.async_copy(table_hbm.at[idx_v.at[b]], rows[b], gsem[b])

    def body(g, carry):
      c0 = g * _NB
      for b in range(_NB):
        c = c0 + b
        pltpu.make_async_copy(table_hbm.at[idx_v.at[c]], rows[b], gsem[b]).wait()
        pltpu.async_copy(rows[b], out_hbm.at[pl.ds(base + c * _CH, _CH)], ssem[b])
      for b in range(_NB):
        c = c0 + b

        @pl.when(c + _NB < _NCH)
        def _(b=b, c=c):
          pltpu.make_async_copy(
              rows[b], out_hbm.at[pl.ds(base + c * _CH, _CH)], ssem[b]).wait()
          pltpu.async_copy(table_hbm.at[idx_v.at[c + _NB]], rows[b], gsem[b])

      return carry

    lax.fori_loop(0, _NG, body, 0)

    for b in range(_NB):
      c = _NCH - _NB + b
      pltpu.make_async_copy(
          rows[b], out_hbm.at[pl.ds(base + c * _CH, _CH)], ssem[b]).wait()

  return k(idx, table)


def kernel(position_ids, table):
  idx = position_ids.reshape(_NW, _NCH, _CH).astype(jnp.int32)
  out = _sc_gather(idx, table)
  return out.reshape(position_ids.shape + (table.shape[1],))


# speedup vs baseline: 1.0014x; 1.0014x over previous
import functools

import jax
import jax.numpy as jnp
from jax import lax
from jax.experimental import pallas as pl
from jax.experimental.pallas import tpu as pltpu
from jax.experimental.pallas import tpu_sc as plsc

_B = 32768
_D = 1024
_NC = 2
_NS = 16
_NW = _NC * _NS
_BPW = _B // _NW
_CH = 16
_NCH = _BPW // _CH
_NB = 7
_NGF = _NCH // _NB
_REM = _NCH - _NGF * _NB


@jax.jit
def _sc_gather(idx, table):
  mesh = plsc.VectorSubcoreMesh(core_axis_name="c", subcore_axis_name="s")

  @functools.partial(
      pl.kernel,
      out_type=jax.ShapeDtypeStruct((_B, _D), jnp.float32),
      mesh=mesh,
      scratch_types=[
          pltpu.VMEM((_NCH, _CH), jnp.int32),
      ]
      + [pltpu.VMEM((_CH, _D), jnp.float32)] * _NB
      + [pltpu.SemaphoreType.DMA] * (2 * _NB),
  )
  def k(idx_hbm, table_hbm, out_hbm, idx_v, *bufs):
    rows = bufs[:_NB]
    gsem = bufs[_NB:2 * _NB]
    ssem = bufs[2 * _NB:]
    wid = lax.axis_index("s") * _NC + lax.axis_index("c")
    base = wid * _BPW
    pltpu.sync_copy(idx_hbm.at[wid], idx_v)

    for b in range(_NB):
      pltpu.async_copy(table_hbm.at[idx_v.at[b]], rows[b], gsem[b])

    def body(g, carry):
      c0 = g * _NB
      for b in range(_NB):
        c = c0 + b
        pltpu.make_async_copy(table_hbm.at[idx_v.at[c]], rows[b], gsem[b]).wait()
        pltpu.async_copy(rows[b], out_hbm.at[pl.ds(base + c * _CH, _CH)], ssem[b])
      for b in range(_NB):
        c = c0 + b

        @pl.when(c + _NB < _NCH)
        def _(b=b, c=c):
          pltpu.make_async_copy(
              rows[b], out_hbm.at[pl.ds(base + c * _CH, _CH)], ssem[b]).wait()
          pltpu.async_copy(table_hbm.at[idx_v.at[c + _NB]], rows[b], gsem[b])

      return carry

    lax.fori_loop(0, _NGF, body, 0)

    for r in range(_REM):
      c = _NGF * _NB + r
      b = c % _NB
      pltpu.make_async_copy(table_hbm.at[idx_v.at[c]], rows[b], gsem[b]).wait()
      pltpu.async_copy(rows[b], out_hbm.at[pl.ds(base + c * _CH, _CH)], ssem[b])

    for t in range(_NCH - _NB, _NCH):
      b = t % _NB
      pltpu.make_async_copy(
          rows[b], out_hbm.at[pl.ds(base + t * _CH, _CH)], ssem[b]).wait()

  return k(idx, table)


def kernel(position_ids, table):
  idx = position_ids.reshape(_NW, _NCH, _CH).astype(jnp.int32)
  out = _sc_gather(idx, table)
  return out.reshape(position_ids.shape + (table.shape[1],))
